# Initial kernel scaffold; baseline (speedup 1.0000x reference)
import jax
import jax.numpy as jnp
from jax.experimental import pallas as pl


def kernel(feature, edge_index, W1, b1, W2, b2, Ws1, bs1, Ws2, bs2):
    raise NotImplementedError("write your pallas kernel here")


# speedup vs baseline: 3.7112x; 3.7112x over previous
import functools

import jax
import jax.numpy as jnp
import numpy as np
from jax import lax
from jax.experimental import pallas as pl
from jax.experimental.pallas import tpu as pltpu
from jax.experimental.pallas import tpu_sc as plsc

N = 10000
E = 320000
DIN = 128
DOUT = 512
K = 5000

NP = 10240
NPAD_ROWS = 64
EROWS = 2560
EP = EROWS * 128
NC = 2
NS = 16
NW = NC * NS
ROWS_SPLIT = EROWS // NW
ROWS_ALL = EROWS // NS
NODES_PER_TILE = NP // NS

ACC_ROWS = 5248
PASS0 = 5120
DISCARD = 5247

_MESH = plsc.VectorSubcoreMesh(core_axis_name="c", subcore_axis_name="s",
                               num_cores=NC, num_subcores=NS)
_F32 = jnp.float32
_I32 = jnp.int32


def _fill(ref, n16, val):
    v = jnp.full((16,), val, dtype=ref.dtype)
    def body(i, _):
        ref[pl.ds(i * 16, 16)] = v
        return 0
    lax.fori_loop(0, n16, body, 0)


def _zero_zbuf(zbuf):
    z = jnp.zeros((16,), dtype=zbuf.dtype)
    def body(r, _):
        for l in range(8):
            zbuf[r, pl.ds(l * 16, 16)] = z
        return 0
    lax.fori_loop(0, 64, body, 0)


def _sc_degrees_body(src_hbm, dst_hbm, out_hbm, idx2, ones_v, zrow, acc_o, acc_i):
    c = lax.axis_index("c")
    s = lax.axis_index("s")
    _fill(ones_v, 8, 1.0)
    _fill(zrow, NODES_PER_TILE // 16, 0.0)
    pltpu.sync_copy(zrow, acc_o.at[pl.ds(s * NODES_PER_TILE, NODES_PER_TILE)])
    pltpu.sync_copy(zrow, acc_i.at[pl.ds(s * NODES_PER_TILE, NODES_PER_TILE)])
    plsc.subcore_barrier()
    row0 = (c * NS + s) * ROWS_SPLIT
    for arr, acc in ((src_hbm, acc_o), (dst_hbm, acc_i)):
        pltpu.sync_copy(arr.at[pl.ds(row0, ROWS_SPLIT), :], idx2)
        def body(r, _):
            pltpu.sync_copy(ones_v, acc.at[idx2.at[r]], add=True)
            return 0
        lax.fori_loop(0, ROWS_SPLIT, body, 0)
    plsc.subcore_barrier()
    sl = pl.ds(s * NODES_PER_TILE, NODES_PER_TILE)
    base = c * 2 * NP + s * NODES_PER_TILE
    pltpu.sync_copy(acc_o.at[sl], out_hbm.at[pl.ds(base, NODES_PER_TILE)])
    pltpu.sync_copy(acc_i.at[sl], out_hbm.at[pl.ds(base + NP, NODES_PER_TILE)])


_sc_degrees = pl.kernel(
    _sc_degrees_body,
    out_type=jax.ShapeDtypeStruct((2 * NC * NP,), _F32),
    mesh=_MESH,
    scratch_types=[
        pltpu.VMEM((ROWS_SPLIT, 128), _I32),
        pltpu.VMEM((128,), _F32),
        pltpu.VMEM((NODES_PER_TILE,), _F32),
        pltpu.VMEM_SHARED((NP,), _F32),
        pltpu.VMEM_SHARED((NP,), _F32),
    ],
)


def _sc_agg_body(C, src_hbm, dst_hbm, x_hbm, out_hbm,
                 src2, dst2, rows0, rows1, idxb0, idxb1, dstb0, dstb1,
                 zbuf, acc, sem0, sem1):
    c = lax.axis_index("c")
    s = lax.axis_index("s")
    if C == 1:
        R = ROWS_SPLIT
        row0 = (c * NS + s) * ROWS_SPLIT
    else:
        R = ROWS_ALL
        row0 = s * ROWS_ALL
    _zero_zbuf(zbuf)
    pltpu.sync_copy(src_hbm.at[pl.ds(row0, R), :], src2)
    pltpu.sync_copy(dst_hbm.at[pl.ds(row0, R), :], dst2)

    for jj in range(1 if C == 1 else 2):
        chunk = jj if C == 1 else c * 2 + jj
        for p in range(2):
            base = 0 if p == 0 else PASS0
            zrows = ACC_ROWS // NS
            for zi in range(6):
                nz = 64 if zi < 5 else 8
                pltpu.sync_copy(zbuf.at[pl.ds(0, nz), :],
                                acc.at[pl.ds(s * zrows + zi * 64, nz), :])
            plsc.subcore_barrier()

            def fill_idx(idxb, dstb, r):
                def b(l, _):
                    sl16 = pl.ds(l * 16, 16)
                    if C == 1:
                        idxb[sl16] = src2[r, sl16]
                    else:
                        idxb[sl16] = src2[r, sl16] + chunk * NP
                    d = dst2[r, sl16] - base
                    if p == 0:
                        ok = d < PASS0
                    else:
                        ok = d >= 0
                    dstb[sl16] = jnp.where(ok, d, DISCARD)
                    return 0
                lax.fori_loop(0, 8, b, 0)

            def pair_body(q, _):
                r0 = q * 2
                r1 = r0 + 1
                fill_idx(idxb0, dstb0, r0)
                d0 = pltpu.async_copy(x_hbm.at[idxb0], rows0, sem0)
                fill_idx(idxb1, dstb1, r1)
                d1 = pltpu.async_copy(x_hbm.at[idxb1], rows1, sem1)
                d0.wait()
                pltpu.sync_copy(rows0, acc.at[dstb0], add=True)
                d1.wait()
                pltpu.sync_copy(rows1, acc.at[dstb1], add=True)
                return 0
            lax.fori_loop(0, R // 2, pair_body, 0)

            plsc.subcore_barrier()
            out_view = out_hbm.at[c] if C == 1 else out_hbm.at[chunk]
            rows_per_tile = PASS0 // NS
            sla = pl.ds(s * rows_per_tile, rows_per_tile)
            slo = pl.ds(p * PASS0 + s * rows_per_tile, rows_per_tile)
            pltpu.sync_copy(acc.at[sla, :], out_view.at[slo, :])
            plsc.subcore_barrier()


def _make_sc_agg(C):
    R = ROWS_SPLIT if C == 1 else ROWS_ALL
    n_out = 2 if C == 1 else 4
    return pl.kernel(
        functools.partial(_sc_agg_body, C),
        out_type=jax.ShapeDtypeStruct((n_out, NP, 128), _F32),
        mesh=_MESH,
        scratch_types=[
            pltpu.VMEM((R, 128), _I32),
            pltpu.VMEM((R, 128), _I32),
            pltpu.VMEM((128, 128), _F32),
            pltpu.VMEM((128, 128), _F32),
            pltpu.VMEM((128,), _I32),
            pltpu.VMEM((128,), _I32),
            pltpu.VMEM((128,), _I32),
            pltpu.VMEM((128,), _I32),
            pltpu.VMEM((64, 128), _F32),
            pltpu.VMEM_SHARED((ACC_ROWS, 128), _F32),
            pltpu.SemaphoreType.DMA,
            pltpu.SemaphoreType.DMA,
        ],
    )


_sc_agg1 = _make_sc_agg(1)
_sc_agg4 = _make_sc_agg(4)


def _sc_scalar_agg_body(src_hbm, dst_hbm, hs_hbm, out_hbm,
                        src2, dst2, valbuf, zrow, acc, sem):
    c = lax.axis_index("c")
    s = lax.axis_index("s")
    _fill(zrow, NODES_PER_TILE // 16, 0.0)
    pltpu.sync_copy(zrow, acc.at[pl.ds(s * NODES_PER_TILE, NODES_PER_TILE)])
    row0 = (c * NS + s) * ROWS_SPLIT
    pltpu.sync_copy(src_hbm.at[pl.ds(row0, ROWS_SPLIT), :], src2)
    pltpu.sync_copy(dst_hbm.at[pl.ds(row0, ROWS_SPLIT), :], dst2)
    plsc.subcore_barrier()

    def body(r, _):
        pltpu.async_copy(hs_hbm.at[src2.at[r]], valbuf, sem).wait()
        pltpu.sync_copy(valbuf, acc.at[dst2.at[r]], add=True)
        return 0
    lax.fori_loop(0, ROWS_SPLIT, body, 0)
    plsc.subcore_barrier()
    sl = pl.ds(s * NODES_PER_TILE, NODES_PER_TILE)
    pltpu.sync_copy(acc.at[sl], out_hbm.at[pl.ds(c * NP + s * NODES_PER_TILE,
                                                 NODES_PER_TILE)])


_sc_scalar_agg = pl.kernel(
    _sc_scalar_agg_body,
    out_type=jax.ShapeDtypeStruct((NC * NP,), _F32),
    mesh=_MESH,
    scratch_types=[
        pltpu.VMEM((ROWS_SPLIT, 128), _I32),
        pltpu.VMEM((ROWS_SPLIT, 128), _I32),
        pltpu.VMEM((128,), _F32),
        pltpu.VMEM((NODES_PER_TILE,), _F32),
        pltpu.VMEM_SHARED((NP,), _F32),
        pltpu.SemaphoreType.DMA,
    ],
)


def _sc_scatter_rows_body(rank_hbm, scaled_hbm, out_hbm, rk, rows):
    c = lax.axis_index("c")
    s = lax.axis_index("s")
    w = c * NS + s
    pltpu.sync_copy(rank_hbm.at[w], rk)

    def body(r, _):
        base = (w * 5 + r) * 64
        pltpu.sync_copy(scaled_hbm.at[pl.ds(base, 64), :], rows)
        pltpu.sync_copy(rows, out_hbm.at[rk.at[r]])
        return 0
    lax.fori_loop(0, 5, body, 0)


_sc_scatter_rows = pl.kernel(
    _sc_scatter_rows_body,
    out_type=jax.ShapeDtypeStruct((NP, DOUT), _F32),
    mesh=_MESH,
    scratch_types=[
        pltpu.VMEM((5, 64), _I32),
        pltpu.VMEM((64, DOUT), _F32),
    ],
)


_RB = 1280
_GRID_R = NP // _RB


def _norms_h0_body(degs_ref, feat_ref, h0_ref, nsrc_ref, ndst_ref):
    do = degs_ref[0, 0] + degs_ref[1, 0]
    di = degs_ref[0, 1] + degs_ref[1, 1]
    ns = lax.rsqrt(jnp.where(do > 0, do, 1.0))
    nd = lax.rsqrt(jnp.where(di > 0, di, 1.0))
    nsrc_ref[...] = ns
    ndst_ref[...] = nd
    h0_ref[...] = feat_ref[...] * ns


def _tc_norms_h0(degs4, feat_p):
    return pl.pallas_call(
        _norms_h0_body,
        grid=(_GRID_R,),
        in_specs=[
            pl.BlockSpec((2, 2, _RB, 1), lambda r: (0, 0, r, 0)),
            pl.BlockSpec((_RB, DIN), lambda r: (r, 0)),
        ],
        out_specs=[
            pl.BlockSpec((_RB, DIN), lambda r: (r, 0)),
            pl.BlockSpec((_RB, 1), lambda r: (r, 0)),
            pl.BlockSpec((_RB, 1), lambda r: (r, 0)),
        ],
        out_shape=[
            jax.ShapeDtypeStruct((NP, DIN), _F32),
            jax.ShapeDtypeStruct((NP, 1), _F32),
            jax.ShapeDtypeStruct((NP, 1), _F32),
        ],
    )(degs4, feat_p)


def _conv1_body(aggp_ref, ndst_ref, nsrc_ref, w_ref, b_ref, out_ref):
    agg = aggp_ref[0] + aggp_ref[1]
    rst = agg * ndst_ref[...]
    z = jnp.dot(rst, w_ref[...], preferred_element_type=_F32)
    out_ref[0] = jax.nn.relu(z + b_ref[...]) * nsrc_ref[...]


def _tc_conv1(agg0p, ndst, nsrc, W1, b1):
    return pl.pallas_call(
        _conv1_body,
        grid=(_GRID_R, 4),
        in_specs=[
            pl.BlockSpec((2, _RB, DIN), lambda r, co: (0, r, 0)),
            pl.BlockSpec((_RB, 1), lambda r, co: (r, 0)),
            pl.BlockSpec((_RB, 1), lambda r, co: (r, 0)),
            pl.BlockSpec((DIN, 128), lambda r, co: (0, co)),
            pl.BlockSpec((1, 128), lambda r, co: (0, co)),
        ],
        out_specs=pl.BlockSpec((1, _RB, 128), lambda r, co: (co, r, 0)),
        out_shape=jax.ShapeDtypeStruct((4, NP, 128), _F32),
    )(agg0p, ndst, nsrc, W1, b1)


def _conv2_body(aggc_ref, ndst_ref, nsrc_ref, w_ref, b_ref, out_ref):
    nd = ndst_ref[...]
    z = jnp.zeros((_RB, 128), _F32)
    for kc in range(4):
        z = z + jnp.dot(aggc_ref[kc] * nd, w_ref[kc * 128:(kc + 1) * 128, :],
                        preferred_element_type=_F32)
    out_ref[0] = jax.nn.relu(z + b_ref[...]) * nsrc_ref[...]


def _tc_conv2(aggc, ndst, nsrc, W2, b2):
    return pl.pallas_call(
        _conv2_body,
        grid=(_GRID_R, 4),
        in_specs=[
            pl.BlockSpec((4, _RB, 128), lambda r, co: (0, r, 0)),
            pl.BlockSpec((_RB, 1), lambda r, co: (r, 0)),
            pl.BlockSpec((_RB, 1), lambda r, co: (r, 0)),
            pl.BlockSpec((DOUT, 128), lambda r, co: (0, co)),
            pl.BlockSpec((1, 128), lambda r, co: (0, co)),
        ],
        out_specs=pl.BlockSpec((1, _RB, 128), lambda r, co: (co, r, 0)),
        out_shape=jax.ShapeDtypeStruct((4, NP, 128), _F32),
    )(aggc, ndst, nsrc, W2, b2)


def _conv_final_body(aggc_ref, ndst_ref, nsrc_ref, w_ref, b_ref, wm_ref,
                     out_ref, hs_ref):
    nd = ndst_ref[...]
    z = jnp.zeros((_RB, DOUT), _F32)
    for kc in range(4):
        z = z + jnp.dot(aggc_ref[kc] * nd, w_ref[kc * 128:(kc + 1) * 128, :],
                        preferred_element_type=_F32)
    o = jax.nn.relu(z + b_ref[...])
    out_ref[...] = o
    y = jnp.dot(o, wm_ref[...], preferred_element_type=_F32)
    hs_ref[...] = y * nsrc_ref[...]


def _tc_conv_final(aggc, ndst, nsrc, W2, b2, Wm):
    return pl.pallas_call(
        _conv_final_body,
        grid=(_GRID_R,),
        in_specs=[
            pl.BlockSpec((4, _RB, 128), lambda r: (0, r, 0)),
            pl.BlockSpec((_RB, 1), lambda r: (r, 0)),
            pl.BlockSpec((_RB, 1), lambda r: (r, 0)),
            pl.BlockSpec((DOUT, DOUT), lambda r: (0, 0)),
            pl.BlockSpec((1, DOUT), lambda r: (0, 0)),
            pl.BlockSpec((DOUT, 1), lambda r: (0, 0)),
        ],
        out_specs=[
            pl.BlockSpec((_RB, DOUT), lambda r: (r, 0)),
            pl.BlockSpec((_RB, 1), lambda r: (r, 0)),
        ],
        out_shape=[
            jax.ShapeDtypeStruct((NP, DOUT), _F32),
            jax.ShapeDtypeStruct((NP, 1), _F32),
        ],
    )(aggc, ndst, nsrc, W2, b2, Wm)


def _score_body(saggp_ref, ndst_ref, bm_ref, score_ref):
    score_ref[...] = (saggp_ref[0] + saggp_ref[1]) * ndst_ref[...] + bm_ref[0, 0]


def _tc_score(saggp, ndst, bm):
    return pl.pallas_call(
        _score_body,
        grid=(1,),
        in_specs=[
            pl.BlockSpec((2, NP, 1), lambda i: (0, 0, 0)),
            pl.BlockSpec((NP, 1), lambda i: (0, 0)),
            pl.BlockSpec((1, 1), lambda i: (0, 0)),
        ],
        out_specs=pl.BlockSpec((NP, 1), lambda i: (0, 0)),
        out_shape=jax.ShapeDtypeStruct((NP, 1), _F32),
    )(saggp, ndst, bm)


_IMIN = np.int32(-2**31)


def _monotone_key(x):
    b = lax.bitcast_convert_type(x, _I32)
    return jnp.where(b >= 0, b, b ^ np.int32(0x7FFFFFFF))


def _rank_final_body(scol_ref, srow_ref, out_ref, rank_ref, scaled_ref, gmax_ref):
    blk = pl.program_id(0)
    i_idx = lax.broadcasted_iota(_I32, (_RB, 1), 0) + blk * _RB
    ki = _monotone_key(scol_ref[...])
    ki = jnp.where(i_idx < N, ki, _IMIN)

    j_idx_all = (lax.broadcasted_iota(_I32, (NP // 128, 128), 0) * 128
                 + lax.broadcasted_iota(_I32, (NP // 128, 128), 1))
    kj_all = _monotone_key(srow_ref[...])
    kj_all = jnp.where(j_idx_all < N, kj_all, _IMIN)

    cnt = jnp.zeros((_RB, 128), _I32)
    for jr in range(NP // 128):
        kj = kj_all[jr:jr + 1, :]
        jj = j_idx_all[jr:jr + 1, :]
        gt = kj > ki
        tie = (kj == ki) & (jj < i_idx)
        cnt = cnt + (gt | tie).astype(_I32)
    rank = jnp.sum(cnt, axis=1, keepdims=True)
    rank_ref[...] = rank

    th = jnp.tanh(scol_ref[...])
    sc = out_ref[...] * th
    scaled_ref[...] = sc
    sel = rank < K
    contrib = jnp.max(jnp.where(sel, sc, -jnp.inf), axis=0, keepdims=True)

    @pl.when(blk == 0)
    def _():
        gmax_ref[...] = jnp.full((1, DOUT), -jnp.inf, _F32)
    gmax_ref[...] = jnp.maximum(gmax_ref[...], contrib)


def _tc_rank_final(score_col, score_row, out):
    return pl.pallas_call(
        _rank_final_body,
        grid=(_GRID_R,),
        in_specs=[
            pl.BlockSpec((_RB, 1), lambda r: (r, 0)),
            pl.BlockSpec((NP // 128, 128), lambda r: (0, 0)),
            pl.BlockSpec((_RB, DOUT), lambda r: (r, 0)),
        ],
        out_specs=[
            pl.BlockSpec((_RB, 1), lambda r: (r, 0)),
            pl.BlockSpec((_RB, DOUT), lambda r: (r, 0)),
            pl.BlockSpec((1, DOUT), lambda r: (0, 0)),
        ],
        out_shape=[
            jax.ShapeDtypeStruct((NP, 1), _I32),
            jax.ShapeDtypeStruct((NP, DOUT), _F32),
            jax.ShapeDtypeStruct((1, DOUT), _F32),
        ],
    )(score_col, score_row, out)


def kernel(feature, edge_index, W1, b1, W2, b2, Ws1, bs1, Ws2, bs2):
    src = edge_index[0].astype(_I32)
    dst = edge_index[1].astype(_I32)
    npad = EP - E
    pad_ids = N + (jnp.arange(npad, dtype=_I32) % NPAD_ROWS)
    src_p = jnp.concatenate([src, pad_ids]).reshape(EROWS, 128)
    dst_p = jnp.concatenate([dst, pad_ids]).reshape(EROWS, 128)
    feat_p = jnp.pad(feature, ((0, NP - N), (0, 0)))

    degs = _sc_degrees(src_p, dst_p)
    h0, nsrc, ndst = _tc_norms_h0(degs.reshape(2, 2, NP, 1), feat_p)
    agg0p = _sc_agg1(src_p, dst_p, h0)
    h1c = _tc_conv1(agg0p, ndst, nsrc, W1, b1.reshape(1, DOUT))
    agg1 = _sc_agg4(src_p, dst_p, h1c.reshape(4 * NP, 128))
    h2c = _tc_conv2(agg1, ndst, nsrc, W2, b2.reshape(1, DOUT))
    agg2 = _sc_agg4(src_p, dst_p, h2c.reshape(4 * NP, 128))

    Wm = (Ws1 + Ws2) * 0.5
    bm = ((bs1 + bs2) * 0.5).reshape(1, 1)
    out, hs = _tc_conv_final(agg2, ndst, nsrc, W2, b2.reshape(1, DOUT), Wm)
    saggp = _sc_scalar_agg(src_p, dst_p, hs.reshape(NP))
    score = _tc_score(saggp.reshape(2, NP, 1), ndst, bm)
    ranks, scaled, gmax = _tc_rank_final(score, score.reshape(NP // 128, 128), out)
    feat_full = _sc_scatter_rows(ranks.reshape(NW, 5, 64), scaled)
    g_out = jnp.concatenate([gmax, gmax], axis=-1)
    return (feat_full[:K], g_out)
